# Initial kernel scaffold; baseline (speedup 1.0000x reference)
import jax
import jax.numpy as jnp
from jax.experimental import pallas as pl


def kernel(inputs_embeds, attention_mask, token_type_ids):
    raise NotImplementedError("write your pallas kernel here")


# speedup vs baseline: 4.0687x; 4.0687x over previous
import jax
import jax.numpy as jnp
import numpy as np
from jax.experimental import pallas as pl

_D_MODEL = 1024
_NUM_BLOCKS = 3
_SEPARATE_CLS = True
_TRUNCATE_SEQ = True
_CLS_TOKEN_TYPE_ID = 2

_ROW_BLOCK = 512


def _pool_pos(pos_id, block_index):
    if _SEPARATE_CLS:
        cls_pos = np.array([-(2 ** block_index) + 1], dtype=pos_id.dtype)
        pooled = pos_id[1:-1] if _TRUNCATE_SEQ else pos_id[1:]
        return np.concatenate([cls_pos, pooled[::2]], 0)
    return pos_id[::2]


def _rel_pos(pos, stride, pooled_pos=None, shift=1):
    if pooled_pos is None:
        pooled_pos = pos
    ref_point = int(pooled_pos[0]) - int(pos[0])
    num_remove = shift * len(pooled_pos)
    max_dist = ref_point + num_remove * stride
    min_dist = int(pooled_pos[0]) - int(pos[-1])
    return np.arange(max_dist, min_dist - 1, -stride, dtype=np.int32)


def _segments(seq_len):
    pos = np.arange(0, seq_len, dtype=np.int32)
    segs = []
    for block_index in range(_NUM_BLOCKS):
        pooling_rel = None
        if block_index != 0:
            pooled_pos = _pool_pos(pos, block_index)
            stride = 2 ** (block_index - 1)
            pooling_rel = _rel_pos(pos, stride, pooled_pos, shift=2)
            pos = pooled_pos
        stride = 2 ** block_index
        rel = _rel_pos(pos, stride)
        segs.append((len(rel), int(rel[0]), stride))
        if pooling_rel is not None:
            segs.append((len(pooling_rel), int(pooling_rel[0]),
                         int(pooling_rel[0] - pooling_rel[1])))
    return segs


def _make_pos_kernel(bounds, r0s, steps, half):
    def _pos_kernel(o_ref):
        i = pl.program_id(0)
        j = pl.program_id(1)
        shape = (_ROW_BLOCK, half)
        t = i * _ROW_BLOCK + jax.lax.broadcasted_iota(jnp.int32, shape, 0)
        k = jax.lax.broadcasted_iota(jnp.int32, shape, 1)
        r = None
        for idx in range(len(r0s)):
            lo = 0 if idx == 0 else bounds[idx - 1]
            val = (r0s[idx] + steps[idx] * lo) - steps[idx] * t
            r = val if r is None else jnp.where(t < lo, r, val)
        inv_freq = 1.0 / jnp.exp(k.astype(jnp.float32)
                                 * (jnp.log(10000.0) / half))
        x = r.astype(jnp.float32) * inv_freq

        @pl.when(j == 0)
        def _():
            o_ref[...] = jnp.sin(x)

        @pl.when(j == 1)
        def _():
            o_ref[...] = jnp.cos(x)

    return _pos_kernel


def _make_mask_kernel(seq_len, row_block):
    def _mask_kernel(tt_ref, ttm_ref, cls_ref):
        i = pl.program_id(1)
        rows = tt_ref[0, 0, pl.ds(i * row_block, row_block)]
        a = rows[:, None]
        b = tt_ref[0, 0, :][None, :]
        ttm_ref[0] = ((a == b) | (a == _CLS_TOKEN_TYPE_ID)
                      | (b == _CLS_TOKEN_TYPE_ID))
        rid = i * row_block + jax.lax.broadcasted_iota(
            jnp.int32, (row_block, seq_len), 0)
        cid = jax.lax.broadcasted_iota(jnp.int32, (row_block, seq_len), 1)
        cls_ref[...] = ((rid > 0) & (cid > 0)).astype(jnp.float32)

    return _mask_kernel


def kernel(inputs_embeds, attention_mask, token_type_ids):
    seq_len = inputs_embeds.shape[1]
    dtype = inputs_embeds.dtype
    half = _D_MODEL // 2

    segs = _segments(seq_len)
    rows = [s[0] for s in segs]
    total = sum(rows)
    bounds = tuple(int(b) for b in np.cumsum(rows))
    r0s = tuple(int(s[1]) for s in segs)
    steps = tuple(int(s[2]) for s in segs)

    grid_rows = total // _ROW_BLOCK
    stacked = pl.pallas_call(
        _make_pos_kernel(bounds, r0s, steps, half),
        grid=(grid_rows, 2),
        out_specs=pl.BlockSpec((_ROW_BLOCK, half), lambda i, j: (i, j)),
        out_shape=jax.ShapeDtypeStruct((total, _D_MODEL), dtype),
    )()

    flat = []
    off = 0
    for n in rows:
        flat.append(stacked[off:off + n])
        off += n

    row_block = 256
    batch = token_type_ids.shape[0]
    ttm, cls_mask = pl.pallas_call(
        _make_mask_kernel(seq_len, row_block),
        grid=(batch, seq_len // row_block),
        in_specs=[pl.BlockSpec((1, 1, seq_len), lambda b, i: (b, 0, 0))],
        out_specs=[
            pl.BlockSpec((1, row_block, seq_len), lambda b, i: (b, i, 0)),
            pl.BlockSpec((row_block, seq_len), lambda b, i: (i, 0)),
        ],
        out_shape=[
            jax.ShapeDtypeStruct((batch, seq_len, seq_len), jnp.bool_),
            jax.ShapeDtypeStruct((seq_len, seq_len), dtype),
        ],
    )(token_type_ids.reshape(batch, 1, seq_len))

    return (*flat, ttm, attention_mask, cls_mask)


# speedup vs baseline: 17.0716x; 4.1959x over previous
import functools

import jax
import jax.numpy as jnp
import numpy as np
from jax.experimental import pallas as pl

_D_MODEL = 1024
_NUM_BLOCKS = 3
_SEPARATE_CLS = True
_TRUNCATE_SEQ = True
_CLS_TOKEN_TYPE_ID = 2

_TILE = 512
_CHUNK = 8


def _pool_pos(pos_id, block_index):
    if _SEPARATE_CLS:
        cls_pos = np.array([-(2 ** block_index) + 1], dtype=pos_id.dtype)
        pooled = pos_id[1:-1] if _TRUNCATE_SEQ else pos_id[1:]
        return np.concatenate([cls_pos, pooled[::2]], 0)
    return pos_id[::2]


def _rel_pos(pos, stride, pooled_pos=None, shift=1):
    if pooled_pos is None:
        pooled_pos = pos
    ref_point = int(pooled_pos[0]) - int(pos[0])
    num_remove = shift * len(pooled_pos)
    max_dist = ref_point + num_remove * stride
    min_dist = int(pooled_pos[0]) - int(pos[-1])
    return np.arange(max_dist, min_dist - 1, -stride, dtype=np.int32)


def _segments(seq_len):
    pos = np.arange(0, seq_len, dtype=np.int32)
    segs = []
    for block_index in range(_NUM_BLOCKS):
        pooling_rel = None
        if block_index != 0:
            pooled_pos = _pool_pos(pos, block_index)
            stride = 2 ** (block_index - 1)
            pooling_rel = _rel_pos(pos, stride, pooled_pos, shift=2)
            pos = pooled_pos
        stride = 2 ** block_index
        rel = _rel_pos(pos, stride)
        segs.append((len(rel), int(rel[0]), stride))
        if pooling_rel is not None:
            segs.append((len(pooling_rel), int(pooling_rel[0]),
                         int(pooling_rel[0] - pooling_rel[1])))
    return segs


def _pos_seg_kernel(o_ref, *, r0, step, half):
    i = pl.program_id(0)
    shape = (_CHUNK, half)
    k = jax.lax.broadcasted_iota(jnp.int32, shape, 1).astype(jnp.float32)
    f = 1.0 / jnp.exp(k * (jnp.log(10000.0) / half))
    row = i * _TILE + jax.lax.broadcasted_iota(jnp.int32, shape, 0)
    x0 = (r0 - step * row).astype(jnp.float32) * f
    s = jnp.sin(x0)
    c = jnp.cos(x0)
    d = (-_CHUNK * step) * f
    cd = jnp.cos(d)
    sd = jnp.sin(d)
    o_ref[0:_CHUNK, 0:half] = s
    o_ref[0:_CHUNK, half:2 * half] = c

    def body(t, carry):
        s, c = carry
        s2 = s * cd + c * sd
        c2 = c * cd - s * sd
        o_ref[pl.ds(t * _CHUNK, _CHUNK), 0:half] = s2
        o_ref[pl.ds(t * _CHUNK, _CHUNK), half:2 * half] = c2
        return s2, c2

    jax.lax.fori_loop(1, _TILE // _CHUNK, body, (s, c))


def _mask_kernel(tt_ref, ttm_ref, cls_ref, *, seq_len, row_block):
    i = pl.program_id(0)
    a = tt_ref[:, 0, pl.ds(i * row_block, row_block)][:, :, None]
    b = tt_ref[:, 0, :][:, None, :]
    ttm_ref[...] = ((a == b) | (a == _CLS_TOKEN_TYPE_ID)
                    | (b == _CLS_TOKEN_TYPE_ID))
    rid = i * row_block + jax.lax.broadcasted_iota(
        jnp.int32, (row_block, seq_len), 0)
    cid = jax.lax.broadcasted_iota(jnp.int32, (row_block, seq_len), 1)
    cls_ref[...] = ((rid > 0) & (cid > 0)).astype(jnp.float32)


def kernel(inputs_embeds, attention_mask, token_type_ids):
    seq_len = inputs_embeds.shape[1]
    dtype = inputs_embeds.dtype
    half = _D_MODEL // 2

    flat = []
    for n_rows, r0, step in _segments(seq_len):
        out = pl.pallas_call(
            functools.partial(_pos_seg_kernel, r0=r0, step=step, half=half),
            grid=(n_rows // _TILE,),
            out_specs=pl.BlockSpec((_TILE, _D_MODEL), lambda i: (i, 0)),
            out_shape=jax.ShapeDtypeStruct((n_rows, _D_MODEL), dtype),
        )()
        flat.append(out)

    row_block = 256
    batch = token_type_ids.shape[0]
    ttm, cls_mask = pl.pallas_call(
        functools.partial(_mask_kernel, seq_len=seq_len, row_block=row_block),
        grid=(seq_len // row_block,),
        in_specs=[pl.BlockSpec((batch, 1, seq_len), lambda i: (0, 0, 0))],
        out_specs=[
            pl.BlockSpec((batch, row_block, seq_len), lambda i: (0, i, 0)),
            pl.BlockSpec((row_block, seq_len), lambda i: (i, 0)),
        ],
        out_shape=[
            jax.ShapeDtypeStruct((batch, seq_len, seq_len), jnp.bool_),
            jax.ShapeDtypeStruct((seq_len, seq_len), dtype),
        ],
    )(token_type_ids.reshape(batch, 1, seq_len))

    return (*flat, ttm, attention_mask, cls_mask)


# speedup vs baseline: 17.4929x; 1.0247x over previous
import jax
import jax.numpy as jnp
import numpy as np
from jax.experimental import pallas as pl

_D_MODEL = 1024
_NUM_BLOCKS = 3
_SEPARATE_CLS = True
_TRUNCATE_SEQ = True
_CLS_TOKEN_TYPE_ID = 2

_TILE = 512
_GROUP = 32


def _pool_pos(pos_id, block_index):
    if _SEPARATE_CLS:
        cls_pos = np.array([-(2 ** block_index) + 1], dtype=pos_id.dtype)
        pooled = pos_id[1:-1] if _TRUNCATE_SEQ else pos_id[1:]
        return np.concatenate([cls_pos, pooled[::2]], 0)
    return pos_id[::2]


def _rel_pos(pos, stride, pooled_pos=None, shift=1):
    if pooled_pos is None:
        pooled_pos = pos
    ref_point = int(pooled_pos[0]) - int(pos[0])
    num_remove = shift * len(pooled_pos)
    max_dist = ref_point + num_remove * stride
    min_dist = int(pooled_pos[0]) - int(pos[-1])
    return np.arange(max_dist, min_dist - 1, -stride, dtype=np.int32)


def _segments(seq_len):
    pos = np.arange(0, seq_len, dtype=np.int32)
    segs = []
    for block_index in range(_NUM_BLOCKS):
        pooling_rel = None
        if block_index != 0:
            pooled_pos = _pool_pos(pos, block_index)
            stride = 2 ** (block_index - 1)
            pooling_rel = _rel_pos(pos, stride, pooled_pos, shift=2)
            pos = pooled_pos
        stride = 2 ** block_index
        rel = _rel_pos(pos, stride)
        segs.append((len(rel), int(rel[0]), stride))
        if pooling_rel is not None:
            segs.append((len(pooling_rel), int(pooling_rel[0]),
                         int(pooling_rel[0] - pooling_rel[1])))
    return segs


def _pos_tile(o_ref, tile, r0, step, half):
    shape = (_GROUP, half)
    k = jax.lax.broadcasted_iota(jnp.int32, shape, 1).astype(jnp.float32)
    f = 1.0 / jnp.exp(k * (jnp.log(10000.0) / half))
    row = tile * _TILE + jax.lax.broadcasted_iota(jnp.int32, shape, 0)
    x0 = (r0 - step * row).astype(jnp.float32) * f
    s = jnp.sin(x0)
    c = jnp.cos(x0)
    d = (-_GROUP * step) * f
    cd = jnp.cos(d)
    sd = jnp.sin(d)
    o_ref[0:_GROUP, 0:half] = s
    o_ref[0:_GROUP, half:2 * half] = c

    def body(t, carry):
        s, c = carry
        s2 = s * cd + c * sd
        c2 = c * cd - s * sd
        o_ref[pl.ds(t * _GROUP, _GROUP), 0:half] = s2
        o_ref[pl.ds(t * _GROUP, _GROUP), half:2 * half] = c2
        return s2, c2

    jax.lax.fori_loop(1, _TILE // _GROUP, body, (s, c))


def _make_fused_kernel(segs, half, seq_len, row_block, mask_tiles):
    def _fused(tt_ref, np0_ref, np1_ref, p1_ref, np2_ref, p2_ref,
               ttm_ref, cls_ref):
        i = pl.program_id(0)
        refs = (np0_ref, np1_ref, p1_ref, np2_ref, p2_ref)
        lo = 0
        for (n_rows, r0, step), ref in zip(segs, refs):
            n_t = n_rows // _TILE

            @pl.when((i >= lo) & (i < lo + n_t))
            def _(ref=ref, r0=r0, step=step, lo=lo):
                _pos_tile(ref, i - lo, r0, step, half)

            lo += n_t

        @pl.when(i < mask_tiles)
        def _():
            a = tt_ref[:, 0, pl.ds(i * row_block, row_block)][:, :, None]
            b = tt_ref[:, 0, :][:, None, :]
            ttm_ref[...] = ((a == b) | (a == _CLS_TOKEN_TYPE_ID)
                            | (b == _CLS_TOKEN_TYPE_ID))
            rid = i * row_block + jax.lax.broadcasted_iota(
                jnp.int32, (row_block, seq_len), 0)
            cid = jax.lax.broadcasted_iota(jnp.int32, (row_block, seq_len), 1)
            cls_ref[...] = ((rid > 0) & (cid > 0)).astype(jnp.float32)

    return _fused


def kernel(inputs_embeds, attention_mask, token_type_ids):
    seq_len = inputs_embeds.shape[1]
    dtype = inputs_embeds.dtype
    half = _D_MODEL // 2
    batch = token_type_ids.shape[0]

    segs = _segments(seq_len)
    seg_tiles = [n // _TILE for n, _, _ in segs]
    grid = sum(seg_tiles)
    row_block = 256
    mask_tiles = seq_len // row_block

    def _seg_map(lo, n_t):
        return lambda i: (jnp.clip(i - lo, 0, n_t - 1), 0)

    seg_specs = []
    lo = 0
    for n_t in seg_tiles:
        seg_specs.append(
            pl.BlockSpec((_TILE, _D_MODEL), _seg_map(lo, n_t)))
        lo += n_t

    out = pl.pallas_call(
        _make_fused_kernel(segs, half, seq_len, row_block, mask_tiles),
        grid=(grid,),
        in_specs=[pl.BlockSpec((batch, 1, seq_len), lambda i: (0, 0, 0))],
        out_specs=[
            *seg_specs,
            pl.BlockSpec((batch, row_block, seq_len),
                         lambda i: (0, jnp.clip(i, 0, mask_tiles - 1), 0)),
            pl.BlockSpec((row_block, seq_len),
                         lambda i: (jnp.clip(i, 0, mask_tiles - 1), 0)),
        ],
        out_shape=[
            *[jax.ShapeDtypeStruct((n, _D_MODEL), dtype)
              for n, _, _ in segs],
            jax.ShapeDtypeStruct((batch, seq_len, seq_len), jnp.bool_),
            jax.ShapeDtypeStruct((seq_len, seq_len), dtype),
        ],
    )(token_type_ids.reshape(batch, 1, seq_len))

    np0, np1, p1, np2, p2, ttm, cls_mask = out
    return (np0, np1, p1, np2, p2, ttm, attention_mask, cls_mask)
